# Initial kernel scaffold; baseline (speedup 1.0000x reference)
import jax
import jax.numpy as jnp
from jax.experimental import pallas as pl


def kernel(x, edge_index, edge_weight, edge_attr, W_f1, b_f1, W_f2, b_f2, W_lin1, W_lin2, b_lin2, W_out, b_out):
    raise NotImplementedError("write your pallas kernel here")


# speedup vs baseline: 1.7306x; 1.7306x over previous
import functools

import jax
import jax.numpy as jnp
import numpy as np
from jax import lax
from jax.experimental import pallas as pl
from jax.experimental.pallas import tpu as pltpu
from jax.experimental.pallas import tpu_sc as plsc

N_NODES = 10000
N_EDGES = 320000
HIDDEN = 128
NUM_RBF = 16
CUTOFF = 5.0

_NC = 2
_NS = 16
_NW = _NC * _NS
_EPW = N_EDGES // _NW
_W = 80
_NCHUNK = _EPW // _W
_RPS = 624
_TAIL = N_NODES - _NS * _RPS
_TAIL_OFF = _NS * _RPS

_sc_mesh = plsc.VectorSubcoreMesh(core_axis_name="c", subcore_axis_name="s")


def _lin1_body(x_ref, w_ref, o_ref):
    o_ref[...] = jnp.dot(x_ref[...], w_ref[...],
                         preferred_element_type=jnp.float32)


def _modulate_body(ea_ref, ew_ref, g_ref, wf1_ref, bf1_ref, wf2_ref, bf2_ref,
                   o_ref):
    t = jnp.tanh(jnp.dot(ea_ref[...], wf1_ref[...],
                         preferred_element_type=jnp.float32) + bf1_ref[...])
    f = jnp.dot(t, wf2_ref[...],
                preferred_element_type=jnp.float32) + bf2_ref[...]
    w = ew_ref[...]
    c = 0.5 * (jnp.cos(w * (np.pi / CUTOFF)) + 1.0)
    c = c * (w < CUTOFF).astype(jnp.float32)
    o_ref[...] = g_ref[...] * f * c


def _out_body(p_ref, wl2_ref, bl2_ref, wo_ref, bo_ref, o_ref):
    a = p_ref[0] + p_ref[1]
    h = jnp.tanh(jnp.dot(a, wl2_ref[...],
                         preferred_element_type=jnp.float32) + bl2_ref[...])
    o_ref[...] = jnp.dot(h, wo_ref[...],
                         preferred_element_type=jnp.float32) + bo_ref[...]


@functools.partial(
    pl.kernel,
    mesh=_sc_mesh,
    out_type=jax.ShapeDtypeStruct((N_EDGES, HIDDEN), jnp.float32),
    scratch_types=[
        pltpu.VMEM((_W,), jnp.int32),
        pltpu.VMEM((_W, HIDDEN), jnp.float32),
        pltpu.SemaphoreType.DMA,
    ],
)
def _sc_gather(h_hbm, src_hbm, out_hbm, idx_v, rows_v, sem):
    wid = lax.axis_index("s") * _NC + lax.axis_index("c")
    base = wid * _EPW

    @pl.loop(0, _NCHUNK)
    def _(c):
        off = base + c * _W
        pltpu.sync_copy(src_hbm.at[pl.ds(off, _W)], idx_v)
        pltpu.async_copy(h_hbm.at[idx_v], rows_v, sem).wait()
        pltpu.sync_copy(rows_v, out_hbm.at[pl.ds(off, _W)])


@functools.partial(
    pl.kernel,
    mesh=_sc_mesh,
    out_type=jax.ShapeDtypeStruct((_NC, N_NODES, HIDDEN), jnp.float32),
    scratch_types=[
        pltpu.VMEM((_W,), jnp.int32),
        pltpu.VMEM((_W, HIDDEN), jnp.float32),
        pltpu.VMEM_SHARED((N_NODES, HIDDEN), jnp.float32),
        pltpu.SemaphoreType.DMA,
    ],
)
def _sc_scatter(msg_hbm, dst_hbm, zeros_hbm, out_hbm, idx_v, rows_v, agg_sp,
                sem):
    cid = lax.axis_index("c")
    sid = lax.axis_index("s")
    pltpu.sync_copy(zeros_hbm.at[pl.ds(sid * _RPS, _RPS)],
                    agg_sp.at[pl.ds(sid * _RPS, _RPS)])

    @pl.when(sid == _NS - 1)
    def _():
        pltpu.sync_copy(zeros_hbm.at[pl.ds(_TAIL_OFF, _TAIL)],
                        agg_sp.at[pl.ds(_TAIL_OFF, _TAIL)])

    plsc.subcore_barrier()

    wid = sid * _NC + cid
    base = wid * _EPW

    @pl.loop(0, _NCHUNK)
    def _(c):
        off = base + c * _W
        pltpu.sync_copy(dst_hbm.at[pl.ds(off, _W)], idx_v)
        pltpu.sync_copy(msg_hbm.at[pl.ds(off, _W)], rows_v)
        pltpu.sync_copy(rows_v, agg_sp.at[idx_v], add=True)

    plsc.subcore_barrier()
    pltpu.sync_copy(agg_sp.at[pl.ds(sid * _RPS, _RPS)],
                    out_hbm.at[cid, pl.ds(sid * _RPS, _RPS)])

    @pl.when(sid == _NS - 1)
    def _():
        pltpu.sync_copy(agg_sp.at[pl.ds(_TAIL_OFF, _TAIL)],
                        out_hbm.at[cid, pl.ds(_TAIL_OFF, _TAIL)])


def kernel(x, edge_index, edge_weight, edge_attr,
           W_f1, b_f1, W_f2, b_f2, W_lin1, W_lin2, b_lin2, W_out, b_out):
    src = edge_index[0]
    dst = edge_index[1]
    ew = edge_weight.reshape(N_EDGES, 1)
    bf1 = b_f1.reshape(1, HIDDEN)
    bf2 = b_f2.reshape(1, HIDDEN)
    bl2 = b_lin2.reshape(1, HIDDEN)
    bo = b_out.reshape(1, HIDDEN)
    zeros = jnp.zeros((N_NODES, HIDDEN), jnp.float32)

    h = pl.pallas_call(
        _lin1_body,
        out_shape=jax.ShapeDtypeStruct((N_NODES, HIDDEN), jnp.float32),
    )(x, W_lin1)

    g = _sc_gather(h, src)

    BE = 8000
    nblk = N_EDGES // BE
    msg = pl.pallas_call(
        _modulate_body,
        grid=(nblk,),
        in_specs=[
            pl.BlockSpec((BE, NUM_RBF), lambda i: (i, 0)),
            pl.BlockSpec((BE, 1), lambda i: (i, 0)),
            pl.BlockSpec((BE, HIDDEN), lambda i: (i, 0)),
            pl.BlockSpec((NUM_RBF, HIDDEN), lambda i: (0, 0)),
            pl.BlockSpec((1, HIDDEN), lambda i: (0, 0)),
            pl.BlockSpec((HIDDEN, HIDDEN), lambda i: (0, 0)),
            pl.BlockSpec((1, HIDDEN), lambda i: (0, 0)),
        ],
        out_specs=pl.BlockSpec((BE, HIDDEN), lambda i: (i, 0)),
        out_shape=jax.ShapeDtypeStruct((N_EDGES, HIDDEN), jnp.float32),
    )(edge_attr, ew, g, W_f1, bf1, W_f2, bf2)

    parts = _sc_scatter(msg, dst, zeros)

    out = pl.pallas_call(
        _out_body,
        out_shape=jax.ShapeDtypeStruct((N_NODES, HIDDEN), jnp.float32),
    )(parts, W_lin2, bl2, W_out, bo)
    return out


# speedup vs baseline: 1.9530x; 1.1285x over previous
import functools

import jax
import jax.numpy as jnp
import numpy as np
from jax import lax
from jax.experimental import pallas as pl
from jax.experimental.pallas import tpu as pltpu
from jax.experimental.pallas import tpu_sc as plsc

N_NODES = 10000
N_EDGES = 320000
HIDDEN = 128
NUM_RBF = 16
CUTOFF = 5.0

_NC = 2
_NS = 16
_NW = _NC * _NS
_EPW = N_EDGES // _NW
_W = 80
_NCHUNK = _EPW // _W
_RPS = 624
_TAIL = N_NODES - _NS * _RPS
_TAIL_OFF = _NS * _RPS

_sc_mesh = plsc.VectorSubcoreMesh(core_axis_name="c", subcore_axis_name="s")


def _lin1_body(x_ref, w_ref, o_ref):
    o_ref[...] = jnp.dot(x_ref[...], w_ref[...],
                         preferred_element_type=jnp.float32)


def _modulate_body(ea_ref, ew_ref, g_ref, wf1_ref, bf1_ref, wf2_ref, bf2_ref,
                   o_ref):
    t = jnp.tanh(jnp.dot(ea_ref[...], wf1_ref[...],
                         preferred_element_type=jnp.float32) + bf1_ref[...])
    f = jnp.dot(t, wf2_ref[...],
                preferred_element_type=jnp.float32) + bf2_ref[...]
    w = ew_ref[...]
    c = 0.5 * (jnp.cos(w * (np.pi / CUTOFF)) + 1.0)
    c = c * (w < CUTOFF).astype(jnp.float32)
    o_ref[...] = g_ref[...] * f * c


def _out_body(p_ref, wl2_ref, bl2_ref, wo_ref, bo_ref, o_ref):
    a = p_ref[0] + p_ref[1]
    h = jnp.tanh(jnp.dot(a, wl2_ref[...],
                         preferred_element_type=jnp.float32) + bl2_ref[...])
    o_ref[...] = jnp.dot(h, wo_ref[...],
                         preferred_element_type=jnp.float32) + bo_ref[...]


@functools.partial(
    pl.kernel,
    mesh=_sc_mesh,
    out_type=jax.ShapeDtypeStruct((N_EDGES, HIDDEN), jnp.float32),
    scratch_types=[
        pltpu.VMEM((_EPW,), jnp.int32),
        pltpu.VMEM((_W, HIDDEN), jnp.float32),
        pltpu.VMEM((_W, HIDDEN), jnp.float32),
        pltpu.SemaphoreType.DMA,
        pltpu.SemaphoreType.DMA,
        pltpu.SemaphoreType.DMA,
        pltpu.SemaphoreType.DMA,
    ],
)
def _sc_gather(h_hbm, src_hbm, out_hbm, idx_v, r0, r1, g0, g1, w0, w1):
    wid = lax.axis_index("s") * _NC + lax.axis_index("c")
    base = wid * _EPW
    pltpu.sync_copy(src_hbm.at[pl.ds(base, _EPW)], idx_v)

    def _gather(c, buf, sem):
        pltpu.make_async_copy(
            h_hbm.at[idx_v.at[pl.ds(c * _W, _W)]], buf, sem).start()

    def _wb(c, buf, sem):
        pltpu.make_async_copy(
            buf, out_hbm.at[pl.ds(base + c * _W, _W)], sem).start()

    def _gwait(buf, sem):
        pltpu.make_async_copy(h_hbm.at[idx_v.at[pl.ds(0, _W)]], buf, sem).wait()

    def _wwait(c, buf, sem):
        pltpu.make_async_copy(
            buf, out_hbm.at[pl.ds(base + c * _W, _W)], sem).wait()

    _gather(0, r0, g0)
    _gather(1, r1, g1)

    @pl.loop(0, _NCHUNK - 1, step=2)
    def _(c):
        _gwait(r0, g0)
        _wb(c, r0, w0)
        _wwait(c, r0, w0)
        _gather(c + 2, r0, g0)
        _gwait(r1, g1)
        _wb(c + 1, r1, w1)
        _wwait(c + 1, r1, w1)

        @pl.when(c + 3 < _NCHUNK)
        def _():
            _gather(c + 3, r1, g1)

    _gwait(r0, g0)
    _wb(_NCHUNK - 1, r0, w0)
    _wwait(_NCHUNK - 1, r0, w0)


@functools.partial(
    pl.kernel,
    mesh=_sc_mesh,
    out_type=jax.ShapeDtypeStruct((_NC, N_NODES, HIDDEN), jnp.float32),
    scratch_types=[
        pltpu.VMEM((_NCHUNK, _W), jnp.int32),
        pltpu.VMEM((_W, HIDDEN), jnp.float32),
        pltpu.VMEM((_W, HIDDEN), jnp.float32),
        pltpu.VMEM_SHARED((N_NODES, HIDDEN), jnp.float32),
        pltpu.SemaphoreType.DMA,
        pltpu.SemaphoreType.DMA,
    ],
)
def _sc_scatter(msg_hbm, dst3_hbm, zeros_hbm, out_hbm, idx2_v, m0, m1, agg_sp,
                s0, s1):
    cid = lax.axis_index("c")
    sid = lax.axis_index("s")
    pltpu.sync_copy(zeros_hbm.at[pl.ds(sid * _RPS, _RPS)],
                    agg_sp.at[pl.ds(sid * _RPS, _RPS)])

    wid = sid * _NC + cid
    base = wid * _EPW
    pltpu.sync_copy(dst3_hbm.at[wid], idx2_v)

    @pl.when(sid == _NS - 1)
    def _():
        pltpu.sync_copy(zeros_hbm.at[pl.ds(_TAIL_OFF, _TAIL)],
                        agg_sp.at[pl.ds(_TAIL_OFF, _TAIL)])

    plsc.subcore_barrier()

    def _mload(c, buf, sem):
        pltpu.make_async_copy(
            msg_hbm.at[pl.ds(base + c * _W, _W)], buf, sem).start()

    def _mwait(c, buf, sem):
        pltpu.make_async_copy(
            msg_hbm.at[pl.ds(base + c * _W, _W)], buf, sem).wait()

    _mload(0, m0, s0)
    _mload(1, m1, s1)

    @pl.loop(0, _NCHUNK - 1, step=2)
    def _(c):
        _mwait(c, m0, s0)
        pltpu.sync_copy(m0, agg_sp.at[idx2_v.at[c]], add=True)
        _mload(c + 2, m0, s0)
        _mwait(c + 1, m1, s1)
        pltpu.sync_copy(m1, agg_sp.at[idx2_v.at[c + 1]], add=True)

        @pl.when(c + 3 < _NCHUNK)
        def _():
            _mload(c + 3, m1, s1)

    _mwait(_NCHUNK - 1, m0, s0)
    pltpu.sync_copy(m0, agg_sp.at[idx2_v.at[_NCHUNK - 1]], add=True)

    plsc.subcore_barrier()
    pltpu.sync_copy(agg_sp.at[pl.ds(sid * _RPS, _RPS)],
                    out_hbm.at[cid, pl.ds(sid * _RPS, _RPS)])

    @pl.when(sid == _NS - 1)
    def _():
        pltpu.sync_copy(agg_sp.at[pl.ds(_TAIL_OFF, _TAIL)],
                        out_hbm.at[cid, pl.ds(_TAIL_OFF, _TAIL)])


def kernel(x, edge_index, edge_weight, edge_attr,
           W_f1, b_f1, W_f2, b_f2, W_lin1, W_lin2, b_lin2, W_out, b_out):
    src = edge_index[0]
    dst = edge_index[1]
    ew = edge_weight.reshape(N_EDGES, 1)
    bf1 = b_f1.reshape(1, HIDDEN)
    bf2 = b_f2.reshape(1, HIDDEN)
    bl2 = b_lin2.reshape(1, HIDDEN)
    bo = b_out.reshape(1, HIDDEN)
    zeros = jnp.zeros((N_NODES, HIDDEN), jnp.float32)

    h = pl.pallas_call(
        _lin1_body,
        out_shape=jax.ShapeDtypeStruct((N_NODES, HIDDEN), jnp.float32),
    )(x, W_lin1)

    g = _sc_gather(h, src)

    BE = 8000
    nblk = N_EDGES // BE
    msg = pl.pallas_call(
        _modulate_body,
        grid=(nblk,),
        in_specs=[
            pl.BlockSpec((BE, NUM_RBF), lambda i: (i, 0)),
            pl.BlockSpec((BE, 1), lambda i: (i, 0)),
            pl.BlockSpec((BE, HIDDEN), lambda i: (i, 0)),
            pl.BlockSpec((NUM_RBF, HIDDEN), lambda i: (0, 0)),
            pl.BlockSpec((1, HIDDEN), lambda i: (0, 0)),
            pl.BlockSpec((HIDDEN, HIDDEN), lambda i: (0, 0)),
            pl.BlockSpec((1, HIDDEN), lambda i: (0, 0)),
        ],
        out_specs=pl.BlockSpec((BE, HIDDEN), lambda i: (i, 0)),
        out_shape=jax.ShapeDtypeStruct((N_EDGES, HIDDEN), jnp.float32),
    )(edge_attr, ew, g, W_f1, bf1, W_f2, bf2)

    dst3 = dst.reshape(_NW, _NCHUNK, _W)
    parts = _sc_scatter(msg, dst3, zeros)

    out = pl.pallas_call(
        _out_body,
        out_shape=jax.ShapeDtypeStruct((N_NODES, HIDDEN), jnp.float32),
    )(parts, W_lin2, bl2, W_out, bo)
    return out


# speedup vs baseline: 3.2969x; 1.6881x over previous
import functools

import jax
import jax.numpy as jnp
import numpy as np
from jax import lax
from jax.experimental import pallas as pl
from jax.experimental.pallas import tpu as pltpu
from jax.experimental.pallas import tpu_sc as plsc

N_NODES = 10000
N_EDGES = 320000
HIDDEN = 128
NUM_RBF = 16
CUTOFF = 5.0

_NC = 2
_NS = 16
_NW = _NC * _NS
_EPW = N_EDGES // _NW
_W = 80
_NCHUNK = _EPW // _W
_RPS = 624
_TAIL = N_NODES - _NS * _RPS
_TAIL_OFF = _NS * _RPS

_sc_mesh = plsc.VectorSubcoreMesh(core_axis_name="c", subcore_axis_name="s")


def _lin1_body(x_ref, w_ref, o_ref):
    o_ref[...] = jnp.dot(x_ref[...], w_ref[...],
                         preferred_element_type=jnp.float32)


def _cutoff_body(ew_ref, o_ref):
    w = ew_ref[...]
    c = 0.5 * (jnp.cos(w * (np.pi / CUTOFF)) + 1.0)
    o_ref[...] = c * (w < CUTOFF).astype(jnp.float32)


def _modulate_body(ea_ref, c_ref, g_ref, wf1_ref, bf1_ref, wf2_ref, bf2_ref,
                   o_ref):
    t = jnp.tanh(jnp.dot(ea_ref[...], wf1_ref[...],
                         preferred_element_type=jnp.float32) + bf1_ref[...])
    f = jnp.dot(t, wf2_ref[...],
                preferred_element_type=jnp.float32) + bf2_ref[...]
    o_ref[...] = g_ref[...] * f * c_ref[...]


def _out_body(p_ref, wl2_ref, bl2_ref, wo_ref, bo_ref, o_ref):
    a = p_ref[0] + p_ref[1]
    h = jnp.tanh(jnp.dot(a, wl2_ref[...],
                         preferred_element_type=jnp.float32) + bl2_ref[...])
    o_ref[...] = jnp.dot(h, wo_ref[...],
                         preferred_element_type=jnp.float32) + bo_ref[...]


@functools.partial(
    pl.kernel,
    mesh=_sc_mesh,
    out_type=jax.ShapeDtypeStruct((N_EDGES, HIDDEN), jnp.float32),
    scratch_types=[
        pltpu.VMEM((_EPW,), jnp.int32),
        pltpu.VMEM((_W, HIDDEN), jnp.float32),
        pltpu.VMEM((_W, HIDDEN), jnp.float32),
        pltpu.SemaphoreType.DMA,
        pltpu.SemaphoreType.DMA,
        pltpu.SemaphoreType.DMA,
        pltpu.SemaphoreType.DMA,
    ],
)
def _sc_gather(h_hbm, src_hbm, out_hbm, idx_v, r0, r1, g0, g1, w0, w1):
    wid = lax.axis_index("s") * _NC + lax.axis_index("c")
    base = wid * _EPW
    pltpu.sync_copy(src_hbm.at[pl.ds(base, _EPW)], idx_v)

    def _gather(c, buf, sem):
        pltpu.make_async_copy(
            h_hbm.at[idx_v.at[pl.ds(c * _W, _W)]], buf, sem).start()

    def _wb(c, buf, sem):
        pltpu.make_async_copy(
            buf, out_hbm.at[pl.ds(base + c * _W, _W)], sem).start()

    def _gwait(buf, sem):
        pltpu.make_async_copy(h_hbm.at[idx_v.at[pl.ds(0, _W)]], buf, sem).wait()

    def _wwait(c, buf, sem):
        pltpu.make_async_copy(
            buf, out_hbm.at[pl.ds(base + c * _W, _W)], sem).wait()

    _gather(0, r0, g0)
    _gather(1, r1, g1)

    @pl.loop(0, _NCHUNK - 1, step=2)
    def _(c):
        _gwait(r0, g0)
        _wb(c, r0, w0)
        _wwait(c, r0, w0)
        _gather(c + 2, r0, g0)
        _gwait(r1, g1)
        _wb(c + 1, r1, w1)
        _wwait(c + 1, r1, w1)

        @pl.when(c + 3 < _NCHUNK)
        def _():
            _gather(c + 3, r1, g1)

    _gwait(r0, g0)
    _wb(_NCHUNK - 1, r0, w0)
    _wwait(_NCHUNK - 1, r0, w0)


@functools.partial(
    pl.kernel,
    mesh=_sc_mesh,
    out_type=jax.ShapeDtypeStruct((_NC, N_NODES, HIDDEN), jnp.float32),
    scratch_types=[
        pltpu.VMEM((_NCHUNK, _W), jnp.int32),
        pltpu.VMEM((_W, HIDDEN), jnp.float32),
        pltpu.VMEM((_W, HIDDEN), jnp.float32),
        pltpu.VMEM_SHARED((N_NODES, HIDDEN), jnp.float32),
        pltpu.SemaphoreType.DMA,
        pltpu.SemaphoreType.DMA,
    ],
)
def _sc_scatter(msg_hbm, dst3_hbm, zeros_hbm, out_hbm, idx2_v, m0, m1, agg_sp,
                s0, s1):
    cid = lax.axis_index("c")
    sid = lax.axis_index("s")
    pltpu.sync_copy(zeros_hbm.at[pl.ds(sid * _RPS, _RPS)],
                    agg_sp.at[pl.ds(sid * _RPS, _RPS)])

    wid = sid * _NC + cid
    base = wid * _EPW
    pltpu.sync_copy(dst3_hbm.at[wid], idx2_v)

    @pl.when(sid == _NS - 1)
    def _():
        pltpu.sync_copy(zeros_hbm.at[pl.ds(_TAIL_OFF, _TAIL)],
                        agg_sp.at[pl.ds(_TAIL_OFF, _TAIL)])

    plsc.subcore_barrier()

    def _mload(c, buf, sem):
        pltpu.make_async_copy(
            msg_hbm.at[pl.ds(base + c * _W, _W)], buf, sem).start()

    def _mwait(c, buf, sem):
        pltpu.make_async_copy(
            msg_hbm.at[pl.ds(base + c * _W, _W)], buf, sem).wait()

    _mload(0, m0, s0)
    _mload(1, m1, s1)

    @pl.loop(0, _NCHUNK - 1, step=2)
    def _(c):
        _mwait(c, m0, s0)
        pltpu.sync_copy(m0, agg_sp.at[idx2_v.at[c]], add=True)
        _mload(c + 2, m0, s0)
        _mwait(c + 1, m1, s1)
        pltpu.sync_copy(m1, agg_sp.at[idx2_v.at[c + 1]], add=True)

        @pl.when(c + 3 < _NCHUNK)
        def _():
            _mload(c + 3, m1, s1)

    _mwait(_NCHUNK - 1, m0, s0)
    pltpu.sync_copy(m0, agg_sp.at[idx2_v.at[_NCHUNK - 1]], add=True)

    plsc.subcore_barrier()
    pltpu.sync_copy(agg_sp.at[pl.ds(sid * _RPS, _RPS)],
                    out_hbm.at[cid, pl.ds(sid * _RPS, _RPS)])

    @pl.when(sid == _NS - 1)
    def _():
        pltpu.sync_copy(agg_sp.at[pl.ds(_TAIL_OFF, _TAIL)],
                        out_hbm.at[cid, pl.ds(_TAIL_OFF, _TAIL)])


def kernel(x, edge_index, edge_weight, edge_attr,
           W_f1, b_f1, W_f2, b_f2, W_lin1, W_lin2, b_lin2, W_out, b_out):
    src = edge_index[0]
    dst = edge_index[1]
    ew2d = edge_weight.reshape(N_EDGES // HIDDEN, HIDDEN)
    bf1 = b_f1.reshape(1, HIDDEN)
    bf2 = b_f2.reshape(1, HIDDEN)
    bl2 = b_lin2.reshape(1, HIDDEN)
    bo = b_out.reshape(1, HIDDEN)
    zeros = jnp.zeros((N_NODES, HIDDEN), jnp.float32)

    cdense = pl.pallas_call(
        _cutoff_body,
        out_shape=jax.ShapeDtypeStruct((N_EDGES // HIDDEN, HIDDEN),
                                       jnp.float32),
    )(ew2d)
    ccol = cdense.reshape(N_EDGES, 1)

    h = pl.pallas_call(
        _lin1_body,
        out_shape=jax.ShapeDtypeStruct((N_NODES, HIDDEN), jnp.float32),
    )(x, W_lin1)

    g = _sc_gather(h, src)

    BE = 8000
    nblk = N_EDGES // BE
    msg = pl.pallas_call(
        _modulate_body,
        grid=(nblk,),
        in_specs=[
            pl.BlockSpec((BE, NUM_RBF), lambda i: (i, 0)),
            pl.BlockSpec((BE, 1), lambda i: (i, 0)),
            pl.BlockSpec((BE, HIDDEN), lambda i: (i, 0)),
            pl.BlockSpec((NUM_RBF, HIDDEN), lambda i: (0, 0)),
            pl.BlockSpec((1, HIDDEN), lambda i: (0, 0)),
            pl.BlockSpec((HIDDEN, HIDDEN), lambda i: (0, 0)),
            pl.BlockSpec((1, HIDDEN), lambda i: (0, 0)),
        ],
        out_specs=pl.BlockSpec((BE, HIDDEN), lambda i: (i, 0)),
        out_shape=jax.ShapeDtypeStruct((N_EDGES, HIDDEN), jnp.float32),
    )(edge_attr, ccol, g, W_f1, bf1, W_f2, bf2)

    dst3 = dst.reshape(_NW, _NCHUNK, _W)
    parts = _sc_scatter(msg, dst3, zeros)

    out = pl.pallas_call(
        _out_body,
        out_shape=jax.ShapeDtypeStruct((N_NODES, HIDDEN), jnp.float32),
    )(parts, W_lin2, bl2, W_out, bo)
    return out


# speedup vs baseline: 4.0353x; 1.2240x over previous
import functools

import jax
import jax.numpy as jnp
import numpy as np
from jax import lax
from jax.experimental import pallas as pl
from jax.experimental.pallas import tpu as pltpu
from jax.experimental.pallas import tpu_sc as plsc

N_NODES = 10000
N_EDGES = 320000
HIDDEN = 128
NUM_RBF = 16
CUTOFF = 5.0

_NC = 2
_NS = 16
_NW = _NC * _NS
_EPW = N_EDGES // _NW
_W = 80
_NCHUNK = _EPW // _W
_RPS = 624
_TAIL = N_NODES - _NS * _RPS
_TAIL_OFF = _NS * _RPS

_sc_mesh = plsc.VectorSubcoreMesh(core_axis_name="c", subcore_axis_name="s")

BE = 6400


def _lin1_body(x_ref, w_ref, o_ref):
    o_ref[...] = jnp.dot(x_ref[...], w_ref[...],
                         preferred_element_type=jnp.float32)


def _cutoff_body(ew_ref, o_ref):
    w = ew_ref[...]
    c = 0.5 * (jnp.cos(w * (np.pi / CUTOFF)) + 1.0)
    o_ref[...] = c * (w < CUTOFF).astype(jnp.float32)


def _modulate_body(ea_ref, c_ref, g_ref, wf1_ref, bf1_ref, wf2_ref, bf2_ref,
                   o_ref):
    t = jnp.tanh(jnp.dot(ea_ref[...], wf1_ref[...],
                         preferred_element_type=jnp.float32) + bf1_ref[...])
    f = jnp.dot(t, wf2_ref[...],
                preferred_element_type=jnp.float32) + bf2_ref[...]
    gf = g_ref[...] * f
    gf3 = gf.reshape(BE // HIDDEN, HIDDEN, HIDDEN)
    o_ref[...] = (gf3 * c_ref[0][:, :, None]).reshape(BE, HIDDEN)


def _out_body(p_ref, wl2_ref, bl2_ref, wo_ref, bo_ref, o_ref):
    a = p_ref[0] + p_ref[1]
    h = jnp.tanh(jnp.dot(a, wl2_ref[...],
                         preferred_element_type=jnp.float32) + bl2_ref[...])
    o_ref[...] = jnp.dot(h, wo_ref[...],
                         preferred_element_type=jnp.float32) + bo_ref[...]


@functools.partial(
    pl.kernel,
    mesh=_sc_mesh,
    out_type=jax.ShapeDtypeStruct((N_EDGES, HIDDEN), jnp.float32),
    scratch_types=[
        pltpu.VMEM((_EPW,), jnp.int32),
        pltpu.VMEM((_W, HIDDEN), jnp.float32),
        pltpu.VMEM((_W, HIDDEN), jnp.float32),
        pltpu.SemaphoreType.DMA,
        pltpu.SemaphoreType.DMA,
        pltpu.SemaphoreType.DMA,
        pltpu.SemaphoreType.DMA,
    ],
)
def _sc_gather(h_hbm, src_hbm, out_hbm, idx_v, r0, r1, g0, g1, w0, w1):
    wid = lax.axis_index("s") * _NC + lax.axis_index("c")
    base = wid * _EPW
    pltpu.sync_copy(src_hbm.at[pl.ds(base, _EPW)], idx_v)

    def _gather(c, buf, sem):
        pltpu.make_async_copy(
            h_hbm.at[idx_v.at[pl.ds(c * _W, _W)]], buf, sem).start()

    def _wb(c, buf, sem):
        pltpu.make_async_copy(
            buf, out_hbm.at[pl.ds(base + c * _W, _W)], sem).start()

    def _gwait(buf, sem):
        pltpu.make_async_copy(h_hbm.at[idx_v.at[pl.ds(0, _W)]], buf, sem).wait()

    def _wwait(c, buf, sem):
        pltpu.make_async_copy(
            buf, out_hbm.at[pl.ds(base + c * _W, _W)], sem).wait()

    _gather(0, r0, g0)
    _gather(1, r1, g1)

    @pl.loop(0, _NCHUNK - 1, step=2)
    def _(c):
        _gwait(r0, g0)
        _wb(c, r0, w0)
        _wwait(c, r0, w0)
        _gather(c + 2, r0, g0)
        _gwait(r1, g1)
        _wb(c + 1, r1, w1)
        _wwait(c + 1, r1, w1)

        @pl.when(c + 3 < _NCHUNK)
        def _():
            _gather(c + 3, r1, g1)

    _gwait(r0, g0)
    _wb(_NCHUNK - 1, r0, w0)
    _wwait(_NCHUNK - 1, r0, w0)


@functools.partial(
    pl.kernel,
    mesh=_sc_mesh,
    out_type=jax.ShapeDtypeStruct((_NC, N_NODES, HIDDEN), jnp.float32),
    scratch_types=[
        pltpu.VMEM((_NCHUNK, _W), jnp.int32),
        pltpu.VMEM((_W, HIDDEN), jnp.float32),
        pltpu.VMEM((_W, HIDDEN), jnp.float32),
        pltpu.VMEM_SHARED((N_NODES, HIDDEN), jnp.float32),
        pltpu.SemaphoreType.DMA,
        pltpu.SemaphoreType.DMA,
    ],
)
def _sc_scatter(msg_hbm, dst3_hbm, zeros_hbm, out_hbm, idx2_v, m0, m1, agg_sp,
                s0, s1):
    cid = lax.axis_index("c")
    sid = lax.axis_index("s")
    pltpu.sync_copy(zeros_hbm.at[pl.ds(sid * _RPS, _RPS)],
                    agg_sp.at[pl.ds(sid * _RPS, _RPS)])

    wid = sid * _NC + cid
    base = wid * _EPW
    pltpu.sync_copy(dst3_hbm.at[wid], idx2_v)

    @pl.when(sid == _NS - 1)
    def _():
        pltpu.sync_copy(zeros_hbm.at[pl.ds(_TAIL_OFF, _TAIL)],
                        agg_sp.at[pl.ds(_TAIL_OFF, _TAIL)])

    plsc.subcore_barrier()

    def _mload(c, buf, sem):
        pltpu.make_async_copy(
            msg_hbm.at[pl.ds(base + c * _W, _W)], buf, sem).start()

    def _mwait(c, buf, sem):
        pltpu.make_async_copy(
            msg_hbm.at[pl.ds(base + c * _W, _W)], buf, sem).wait()

    _mload(0, m0, s0)
    _mload(1, m1, s1)

    @pl.loop(0, _NCHUNK - 1, step=2)
    def _(c):
        _mwait(c, m0, s0)
        pltpu.sync_copy(m0, agg_sp.at[idx2_v.at[c]], add=True)
        _mload(c + 2, m0, s0)
        _mwait(c + 1, m1, s1)
        pltpu.sync_copy(m1, agg_sp.at[idx2_v.at[c + 1]], add=True)

        @pl.when(c + 3 < _NCHUNK)
        def _():
            _mload(c + 3, m1, s1)

    _mwait(_NCHUNK - 1, m0, s0)
    pltpu.sync_copy(m0, agg_sp.at[idx2_v.at[_NCHUNK - 1]], add=True)

    plsc.subcore_barrier()
    pltpu.sync_copy(agg_sp.at[pl.ds(sid * _RPS, _RPS)],
                    out_hbm.at[cid, pl.ds(sid * _RPS, _RPS)])

    @pl.when(sid == _NS - 1)
    def _():
        pltpu.sync_copy(agg_sp.at[pl.ds(_TAIL_OFF, _TAIL)],
                        out_hbm.at[cid, pl.ds(_TAIL_OFF, _TAIL)])


def kernel(x, edge_index, edge_weight, edge_attr,
           W_f1, b_f1, W_f2, b_f2, W_lin1, W_lin2, b_lin2, W_out, b_out):
    src = edge_index[0]
    dst = edge_index[1]
    ew2d = edge_weight.reshape(N_EDGES // HIDDEN, HIDDEN)
    bf1 = b_f1.reshape(1, HIDDEN)
    bf2 = b_f2.reshape(1, HIDDEN)
    bl2 = b_lin2.reshape(1, HIDDEN)
    bo = b_out.reshape(1, HIDDEN)
    zeros = jnp.zeros((N_NODES, HIDDEN), jnp.float32)

    cdense = pl.pallas_call(
        _cutoff_body,
        out_shape=jax.ShapeDtypeStruct((N_EDGES // HIDDEN, HIDDEN),
                                       jnp.float32),
    )(ew2d)

    h = pl.pallas_call(
        _lin1_body,
        out_shape=jax.ShapeDtypeStruct((N_NODES, HIDDEN), jnp.float32),
    )(x, W_lin1)

    g = _sc_gather(h, src)

    nblk = N_EDGES // BE
    msg = pl.pallas_call(
        _modulate_body,
        grid=(nblk,),
        in_specs=[
            pl.BlockSpec((BE, NUM_RBF), lambda i: (i, 0)),
            pl.BlockSpec((1, BE // HIDDEN, HIDDEN), lambda i: (i, 0, 0)),
            pl.BlockSpec((BE, HIDDEN), lambda i: (i, 0)),
            pl.BlockSpec((NUM_RBF, HIDDEN), lambda i: (0, 0)),
            pl.BlockSpec((1, HIDDEN), lambda i: (0, 0)),
            pl.BlockSpec((HIDDEN, HIDDEN), lambda i: (0, 0)),
            pl.BlockSpec((1, HIDDEN), lambda i: (0, 0)),
        ],
        out_specs=pl.BlockSpec((BE, HIDDEN), lambda i: (i, 0)),
        out_shape=jax.ShapeDtypeStruct((N_EDGES, HIDDEN), jnp.float32),
    )(edge_attr, cdense.reshape(nblk, BE // HIDDEN, HIDDEN),
      g, W_f1, bf1, W_f2, bf2)

    dst3 = dst.reshape(_NW, _NCHUNK, _W)
    parts = _sc_scatter(msg, dst3, zeros)

    out = pl.pallas_call(
        _out_body,
        out_shape=jax.ShapeDtypeStruct((N_NODES, HIDDEN), jnp.float32),
    )(parts, W_lin2, bl2, W_out, bo)
    return out


# speedup vs baseline: 4.0721x; 1.0091x over previous
import functools

import jax
import jax.numpy as jnp
import numpy as np
from jax import lax
from jax.experimental import pallas as pl
from jax.experimental.pallas import tpu as pltpu
from jax.experimental.pallas import tpu_sc as plsc

N_NODES = 10000
N_EDGES = 320000
HIDDEN = 128
NUM_RBF = 16
CUTOFF = 5.0

_NC = 2
_NS = 16
_NW = _NC * _NS
_EPW = N_EDGES // _NW
_W = 80
_NCHUNK = _EPW // _W
_SUP = 5
_SW = _SUP * _W
_NSUP = _EPW // _SW
_RPS = 624
_TAIL = N_NODES - _NS * _RPS
_TAIL_OFF = _NS * _RPS

_sc_mesh = plsc.VectorSubcoreMesh(core_axis_name="c", subcore_axis_name="s")

BE = 6400


def _lin1_body(x_ref, w_ref, o_ref):
    o_ref[...] = jnp.dot(x_ref[...], w_ref[...],
                         preferred_element_type=jnp.float32)


def _cutoff_body(ew_ref, o_ref):
    w = ew_ref[...]
    c = 0.5 * (jnp.cos(w * (np.pi / CUTOFF)) + 1.0)
    o_ref[...] = c * (w < CUTOFF).astype(jnp.float32)


def _modulate_body(ea_ref, c_ref, g_ref, wf1_ref, bf1_ref, wf2_ref, bf2_ref,
                   o_ref):
    t = jnp.tanh(jnp.dot(ea_ref[...], wf1_ref[...],
                         preferred_element_type=jnp.float32) + bf1_ref[...])
    f = jnp.dot(t, wf2_ref[...],
                preferred_element_type=jnp.float32) + bf2_ref[...]
    gf = g_ref[...] * f
    gf3 = gf.reshape(BE // HIDDEN, HIDDEN, HIDDEN)
    o_ref[...] = (gf3 * c_ref[0][:, :, None]).reshape(BE, HIDDEN)


def _out_body(p_ref, wl2_ref, bl2_ref, wo_ref, bo_ref, o_ref):
    a = p_ref[0] + p_ref[1]
    h = jnp.tanh(jnp.dot(a, wl2_ref[...],
                         preferred_element_type=jnp.float32) + bl2_ref[...])
    o_ref[...] = jnp.dot(h, wo_ref[...],
                         preferred_element_type=jnp.float32) + bo_ref[...]


@functools.partial(
    pl.kernel,
    mesh=_sc_mesh,
    out_type=jax.ShapeDtypeStruct((N_EDGES, HIDDEN), jnp.float32),
    scratch_types=[
        pltpu.VMEM((_EPW,), jnp.int32),
        pltpu.VMEM((_SW, HIDDEN), jnp.float32),
        pltpu.VMEM((_SW, HIDDEN), jnp.float32),
        pltpu.SemaphoreType.DMA,
        pltpu.SemaphoreType.DMA,
        pltpu.SemaphoreType.DMA,
        pltpu.SemaphoreType.DMA,
    ],
)
def _sc_gather(h_hbm, src_hbm, out_hbm, idx_v, r0, r1, g0, g1, w0, w1):
    wid = lax.axis_index("s") * _NC + lax.axis_index("c")
    base = wid * _EPW
    pltpu.sync_copy(src_hbm.at[pl.ds(base, _EPW)], idx_v)

    def _fire(i, buf, sem):
        for b in range(_SUP):
            pltpu.make_async_copy(
                h_hbm.at[idx_v.at[pl.ds(i * _SW + b * _W, _W)]],
                buf.at[pl.ds(b * _W, _W)], sem).start()

    def _drain(i, buf, sem):
        for b in range(_SUP):
            pltpu.make_async_copy(
                h_hbm.at[idx_v.at[pl.ds(i * _SW + b * _W, _W)]],
                buf.at[pl.ds(b * _W, _W)], sem).wait()

    def _wb(i, buf, sem):
        pltpu.make_async_copy(
            buf, out_hbm.at[pl.ds(base + i * _SW, _SW)], sem).start()

    def _wwait(i, buf, sem):
        pltpu.make_async_copy(
            buf, out_hbm.at[pl.ds(base + i * _SW, _SW)], sem).wait()

    _fire(0, r0, g0)
    _fire(1, r1, g1)

    @pl.loop(0, _NSUP - 1, step=2)
    def _(i):
        _drain(i, r0, g0)
        _wb(i, r0, w0)
        _wwait(i, r0, w0)
        _fire(i + 2, r0, g0)
        _drain(i + 1, r1, g1)
        _wb(i + 1, r1, w1)
        _wwait(i + 1, r1, w1)

        @pl.when(i + 3 < _NSUP)
        def _():
            _fire(i + 3, r1, g1)

    _drain(_NSUP - 1, r0, g0)
    _wb(_NSUP - 1, r0, w0)
    _wwait(_NSUP - 1, r0, w0)


@functools.partial(
    pl.kernel,
    mesh=_sc_mesh,
    out_type=jax.ShapeDtypeStruct((_NC, N_NODES, HIDDEN), jnp.float32),
    scratch_types=[
        pltpu.VMEM((_NCHUNK, _W), jnp.int32),
        pltpu.VMEM((_W, HIDDEN), jnp.float32),
        pltpu.VMEM((_W, HIDDEN), jnp.float32),
        pltpu.VMEM_SHARED((N_NODES, HIDDEN), jnp.float32),
        pltpu.SemaphoreType.DMA,
        pltpu.SemaphoreType.DMA,
    ],
)
def _sc_scatter(msg_hbm, dst3_hbm, zeros_hbm, out_hbm, idx2_v, m0, m1, agg_sp,
                s0, s1):
    cid = lax.axis_index("c")
    sid = lax.axis_index("s")
    pltpu.sync_copy(zeros_hbm.at[pl.ds(sid * _RPS, _RPS)],
                    agg_sp.at[pl.ds(sid * _RPS, _RPS)])

    wid = sid * _NC + cid
    base = wid * _EPW
    pltpu.sync_copy(dst3_hbm.at[wid], idx2_v)

    @pl.when(sid == _NS - 1)
    def _():
        pltpu.sync_copy(zeros_hbm.at[pl.ds(_TAIL_OFF, _TAIL)],
                        agg_sp.at[pl.ds(_TAIL_OFF, _TAIL)])

    plsc.subcore_barrier()

    def _mload(c, buf, sem):
        pltpu.make_async_copy(
            msg_hbm.at[pl.ds(base + c * _W, _W)], buf, sem).start()

    def _mwait(c, buf, sem):
        pltpu.make_async_copy(
            msg_hbm.at[pl.ds(base + c * _W, _W)], buf, sem).wait()

    _mload(0, m0, s0)
    _mload(1, m1, s1)

    @pl.loop(0, _NCHUNK - 1, step=2)
    def _(c):
        _mwait(c, m0, s0)
        pltpu.sync_copy(m0, agg_sp.at[idx2_v.at[c]], add=True)
        _mload(c + 2, m0, s0)
        _mwait(c + 1, m1, s1)
        pltpu.sync_copy(m1, agg_sp.at[idx2_v.at[c + 1]], add=True)

        @pl.when(c + 3 < _NCHUNK)
        def _():
            _mload(c + 3, m1, s1)

    _mwait(_NCHUNK - 1, m0, s0)
    pltpu.sync_copy(m0, agg_sp.at[idx2_v.at[_NCHUNK - 1]], add=True)

    plsc.subcore_barrier()
    pltpu.sync_copy(agg_sp.at[pl.ds(sid * _RPS, _RPS)],
                    out_hbm.at[cid, pl.ds(sid * _RPS, _RPS)])

    @pl.when(sid == _NS - 1)
    def _():
        pltpu.sync_copy(agg_sp.at[pl.ds(_TAIL_OFF, _TAIL)],
                        out_hbm.at[cid, pl.ds(_TAIL_OFF, _TAIL)])


def kernel(x, edge_index, edge_weight, edge_attr,
           W_f1, b_f1, W_f2, b_f2, W_lin1, W_lin2, b_lin2, W_out, b_out):
    src = edge_index[0]
    dst = edge_index[1]
    ew2d = edge_weight.reshape(N_EDGES // HIDDEN, HIDDEN)
    bf1 = b_f1.reshape(1, HIDDEN)
    bf2 = b_f2.reshape(1, HIDDEN)
    bl2 = b_lin2.reshape(1, HIDDEN)
    bo = b_out.reshape(1, HIDDEN)
    zeros = jnp.zeros((N_NODES, HIDDEN), jnp.float32)

    cdense = pl.pallas_call(
        _cutoff_body,
        out_shape=jax.ShapeDtypeStruct((N_EDGES // HIDDEN, HIDDEN),
                                       jnp.float32),
    )(ew2d)

    h = pl.pallas_call(
        _lin1_body,
        out_shape=jax.ShapeDtypeStruct((N_NODES, HIDDEN), jnp.float32),
    )(x, W_lin1)

    g = _sc_gather(h, src)

    nblk = N_EDGES // BE
    msg = pl.pallas_call(
        _modulate_body,
        grid=(nblk,),
        in_specs=[
            pl.BlockSpec((BE, NUM_RBF), lambda i: (i, 0)),
            pl.BlockSpec((1, BE // HIDDEN, HIDDEN), lambda i: (i, 0, 0)),
            pl.BlockSpec((BE, HIDDEN), lambda i: (i, 0)),
            pl.BlockSpec((NUM_RBF, HIDDEN), lambda i: (0, 0)),
            pl.BlockSpec((1, HIDDEN), lambda i: (0, 0)),
            pl.BlockSpec((HIDDEN, HIDDEN), lambda i: (0, 0)),
            pl.BlockSpec((1, HIDDEN), lambda i: (0, 0)),
        ],
        out_specs=pl.BlockSpec((BE, HIDDEN), lambda i: (i, 0)),
        out_shape=jax.ShapeDtypeStruct((N_EDGES, HIDDEN), jnp.float32),
    )(edge_attr, cdense.reshape(nblk, BE // HIDDEN, HIDDEN),
      g, W_f1, bf1, W_f2, bf2)

    dst3 = dst.reshape(_NW, _NCHUNK, _W)
    parts = _sc_scatter(msg, dst3, zeros)

    out = pl.pallas_call(
        _out_body,
        out_shape=jax.ShapeDtypeStruct((N_NODES, HIDDEN), jnp.float32),
    )(parts, W_lin2, bl2, W_out, bo)
    return out
